# baseline (speedup 1.0000x reference)
import functools

import jax
import jax.numpy as jnp
from jax import lax
from jax.experimental import pallas as pl
from jax.experimental.pallas import tpu as pltpu
from jax.experimental.pallas import tpu_sc as plsc

N_NODES = 10000
N_EDGES = 320000
CH = 128

NC = 2
NS = 16
NW = NC * NS

K = 128
CHUNKS = 80
CT = NW * CHUNKS
EPAD = CT * K
NPAD = 10240
RPT = NPAD // NS
RB = 1280
NPH = 2
CPP = CHUNKS // NPH
PH = 40
C0 = 160

_mesh = plsc.VectorSubcoreMesh(core_axis_name="c", subcore_axis_name="s")


@functools.partial(
    pl.kernel,
    mesh=_mesh,
    out_type=jax.ShapeDtypeStruct((NC * NPAD,), jnp.float32),
    scratch_types=[
        pltpu.VMEM((CHUNKS, K), jnp.int32),
        pltpu.VMEM((K,), jnp.float32),
        pltpu.VMEM_SHARED((NPAD,), jnp.float32),
    ],
)
def _sc_degree(dst_hbm, zeros_hbm, deg_hbm, idx_v, ones_v, deg_sh):
    c = lax.axis_index("c")
    s = lax.axis_index("s")
    t = c * NS + s
    for i in range(K // 16):
        ones_v[pl.ds(i * 16, 16)] = jnp.ones((16,), jnp.float32)
    pltpu.sync_copy(zeros_hbm.at[pl.ds(s * RPT, RPT)], deg_sh.at[pl.ds(s * RPT, RPT)])
    pltpu.sync_copy(dst_hbm.at[pl.ds(t * CHUNKS, CHUNKS)], idx_v)
    plsc.subcore_barrier()

    def body(j, carry):
        pltpu.sync_copy(ones_v, deg_sh.at[idx_v.at[j]], add=True)
        return carry

    lax.fori_loop(0, CHUNKS, body, 0)
    plsc.subcore_barrier()
    pltpu.sync_copy(deg_sh.at[pl.ds(s * RPT, RPT)],
                    deg_hbm.at[pl.ds(c * NPAD + s * RPT, RPT)])


@functools.partial(
    pl.kernel,
    mesh=_mesh,
    out_type=jax.ShapeDtypeStruct((NPAD, CH), jnp.float32),
    scratch_types=[
        pltpu.VMEM((PH, K), jnp.int32),
        pltpu.VMEM((PH, K), jnp.int32),
        pltpu.VMEM((K, CH), jnp.float32),
        pltpu.VMEM((K, CH), jnp.float32),
        pltpu.VMEM_SHARED((NPAD, CH), jnp.float32),
        pltpu.SemaphoreType.DMA,
        pltpu.SemaphoreType.DMA,
    ],
)
def _sc_edge_acc(y_hbm, src_hbm, dst_hbm, zeros_hbm, acc_hbm,
                 sidx, didx, rows0, rows1, acc_sh, sem0, sem1):
    c = lax.axis_index("c")
    s = lax.axis_index("s")

    @pl.when(c == 0)
    def _():
        pltpu.sync_copy(zeros_hbm.at[pl.ds(s * RPT, RPT)],
                        acc_sh.at[pl.ds(s * RPT, RPT)])
        plsc.subcore_barrier()

        groups = PH // 2
        for ph in range(C0 // PH):
            base = s * C0 + ph * PH
            pltpu.sync_copy(src_hbm.at[pl.ds(base, PH)], sidx)
            pltpu.sync_copy(dst_hbm.at[pl.ds(base, PH)], didx)
            pltpu.make_async_copy(y_hbm.at[sidx.at[0]], rows0, sem0).start()

            def body(g, carry):
                j0 = 2 * g
                pltpu.make_async_copy(y_hbm.at[sidx.at[j0 + 1]], rows1, sem1).start()
                pltpu.make_async_copy(y_hbm.at[sidx.at[j0]], rows0, sem0).wait()
                pltpu.sync_copy(rows0, acc_sh.at[didx.at[j0]], add=True)

                @pl.when(g + 1 < groups)
                def _():
                    pltpu.make_async_copy(y_hbm.at[sidx.at[j0 + 2]], rows0, sem0).start()

                pltpu.make_async_copy(y_hbm.at[sidx.at[j0 + 1]], rows1, sem1).wait()
                pltpu.sync_copy(rows1, acc_sh.at[didx.at[j0 + 1]], add=True)
                return carry

            lax.fori_loop(0, groups, body, 0)
        plsc.subcore_barrier()
        pltpu.sync_copy(acc_sh.at[pl.ds(s * RPT, RPT)],
                        acc_hbm.at[pl.ds(s * RPT, RPT)])


def _mm_body(x_ref, w_ref, d0_ref, d1_ref, y_ref):
    dis = lax.rsqrt(d0_ref[...] + d1_ref[...] + 1.0)
    xw = jnp.dot(x_ref[...], w_ref[...], preferred_element_type=jnp.float32)
    y_ref[...] = xw * dis


def _fin_body(a0_ref, y_ref, d0_ref, d1_ref, b_ref, o_ref):
    dis = lax.rsqrt(d0_ref[...] + d1_ref[...] + 1.0)
    temp = dis * (a0_ref[...] + y_ref[...]) + b_ref[...]
    o_ref[...] = temp + jnp.where(temp >= 0, temp, 0.01 * temp)


def kernel(x, edges, W, b):
    src = edges[0].astype(jnp.int32)
    dst = edges[1].astype(jnp.int32)
    pad = jnp.full((EPAD - N_EDGES,), N_NODES, jnp.int32)
    src2d = jnp.concatenate([src, pad]).reshape(CT, K)
    dst2d = jnp.concatenate([dst, pad]).reshape(CT, K)
    x_pad = jnp.pad(x, ((0, NPAD - N_NODES), (0, 0)))
    zeros_n = jnp.zeros((NPAD,), jnp.float32)
    zeros_nc = jnp.zeros((NPAD, CH), jnp.float32)

    degp = _sc_degree(dst2d, zeros_n)
    d0 = degp[:NPAD].reshape(NPAD, 1)
    d1 = degp[NPAD:].reshape(NPAD, 1)

    y = pl.pallas_call(
        _mm_body,
        grid=(NPAD // RB,),
        in_specs=[
            pl.BlockSpec((RB, CH), lambda i: (i, 0)),
            pl.BlockSpec((CH, CH), lambda i: (0, 0)),
            pl.BlockSpec((RB, 1), lambda i: (i, 0)),
            pl.BlockSpec((RB, 1), lambda i: (i, 0)),
        ],
        out_specs=pl.BlockSpec((RB, CH), lambda i: (i, 0)),
        out_shape=jax.ShapeDtypeStruct((NPAD, CH), jnp.float32),
    )(x_pad, W, d0, d1)

    accp = _sc_edge_acc(y, src2d, dst2d, zeros_nc)

    out_pad = pl.pallas_call(
        _fin_body,
        grid=(NPAD // RB,),
        in_specs=[
            pl.BlockSpec((RB, CH), lambda i: (i, 0)),
            pl.BlockSpec((RB, CH), lambda i: (i, 0)),
            pl.BlockSpec((RB, 1), lambda i: (i, 0)),
            pl.BlockSpec((RB, 1), lambda i: (i, 0)),
            pl.BlockSpec((1, CH), lambda i: (0, 0)),
        ],
        out_specs=pl.BlockSpec((RB, CH), lambda i: (i, 0)),
        out_shape=jax.ShapeDtypeStruct((NPAD, CH), jnp.float32),
    )(accp, y, d0, d1, b.reshape(1, CH))

    return out_pad[:N_NODES]


# speedup vs baseline: 3.6433x; 3.6433x over previous
import functools

import jax
import jax.numpy as jnp
from jax import lax
from jax.experimental import pallas as pl
from jax.experimental.pallas import tpu as pltpu
from jax.experimental.pallas import tpu_sc as plsc

N_NODES = 10000
N_EDGES = 320000
CH = 128

NC = 2
NS = 16
NW = NC * NS

K = 125
CT = 2560
CHUNKS = CT // NW
PH = 40
NPAD = 10240
RPT = NPAD // NS
RBM = 1000

_mesh = plsc.VectorSubcoreMesh(core_axis_name="c", subcore_axis_name="s")


@functools.partial(
    pl.kernel,
    mesh=_mesh,
    out_type=jax.ShapeDtypeStruct((NC * NPAD,), jnp.float32),
    scratch_types=[
        pltpu.VMEM((CHUNKS, K), jnp.int32),
        pltpu.VMEM((128,), jnp.float32),
        pltpu.VMEM((RPT,), jnp.float32),
        pltpu.VMEM_SHARED((NPAD,), jnp.float32),
    ],
)
def _sc_degree(dst_hbm, deg_hbm, idx_v, ones_v, zero_v, deg_sh):
    c = lax.axis_index("c")
    s = lax.axis_index("s")
    t = c * NS + s
    for i in range(128 // 16):
        ones_v[pl.ds(i * 16, 16)] = jnp.ones((16,), jnp.float32)

    def zbody(i, carry):
        zero_v[pl.ds(i * 16, 16)] = jnp.zeros((16,), jnp.float32)
        return carry

    lax.fori_loop(0, RPT // 16, zbody, 0)
    pltpu.sync_copy(zero_v, deg_sh.at[pl.ds(s * RPT, RPT)])
    pltpu.sync_copy(dst_hbm.at[pl.ds(t * CHUNKS, CHUNKS)], idx_v)
    plsc.subcore_barrier()

    def body(j, carry):
        pltpu.sync_copy(ones_v.at[pl.ds(0, K)], deg_sh.at[idx_v.at[j]], add=True)
        return carry

    lax.fori_loop(0, CHUNKS, body, 0)
    plsc.subcore_barrier()
    pltpu.sync_copy(deg_sh.at[pl.ds(s * RPT, RPT)],
                    deg_hbm.at[pl.ds(c * NPAD + s * RPT, RPT)])


@functools.partial(
    pl.kernel,
    mesh=_mesh,
    out_type=jax.ShapeDtypeStruct((NC * NPAD, CH), jnp.float32),
    scratch_types=[
        pltpu.VMEM((PH, K), jnp.int32),
        pltpu.VMEM((PH, K), jnp.int32),
        pltpu.VMEM((128, CH), jnp.float32),
        pltpu.VMEM((128, CH), jnp.float32),
        pltpu.VMEM_SHARED((NPAD, CH), jnp.float32),
        pltpu.SemaphoreType.DMA,
        pltpu.SemaphoreType.DMA,
    ],
)
def _sc_edge_acc(y_hbm, src_hbm, dst_hbm, acc_hbm,
                 sidx, didx, rows0, rows1, acc_sh, sem0, sem1):
    c = lax.axis_index("c")
    s = lax.axis_index("s")
    t = c * NS + s

    def zbody(i, carry):
        for k in range(CH // 16):
            rows0[i, pl.ds(k * 16, 16)] = jnp.zeros((16,), jnp.float32)
        return carry

    lax.fori_loop(0, 128, zbody, 0)
    for r in range(RPT // 128):
        pltpu.sync_copy(rows0, acc_sh.at[pl.ds(s * RPT + r * 128, 128)])
    plsc.subcore_barrier()

    b0 = rows0.at[pl.ds(0, K)]
    b1 = rows1.at[pl.ds(0, K)]
    groups = PH // 2
    for ph in range(CHUNKS // PH):
        base = t * CHUNKS + ph * PH
        pltpu.sync_copy(src_hbm.at[pl.ds(base, PH)], sidx)
        pltpu.sync_copy(dst_hbm.at[pl.ds(base, PH)], didx)
        pltpu.make_async_copy(y_hbm.at[sidx.at[0]], b0, sem0).start()

        def body(g, carry):
            j0 = 2 * g
            pltpu.make_async_copy(y_hbm.at[sidx.at[j0 + 1]], b1, sem1).start()
            pltpu.make_async_copy(y_hbm.at[sidx.at[j0]], b0, sem0).wait()
            pltpu.sync_copy(b0, acc_sh.at[didx.at[j0]], add=True)

            @pl.when(g + 1 < groups)
            def _():
                pltpu.make_async_copy(y_hbm.at[sidx.at[j0 + 2]], b0, sem0).start()

            pltpu.make_async_copy(y_hbm.at[sidx.at[j0 + 1]], b1, sem1).wait()
            pltpu.sync_copy(b1, acc_sh.at[didx.at[j0 + 1]], add=True)
            return carry

        lax.fori_loop(0, groups, body, 0)
    plsc.subcore_barrier()
    pltpu.sync_copy(acc_sh.at[pl.ds(s * RPT, RPT)],
                    acc_hbm.at[pl.ds(c * NPAD + s * RPT, RPT)])


def _mm_body(x_ref, w_ref, d0_ref, d1_ref, y_ref):
    dis = lax.rsqrt(d0_ref[...] + d1_ref[...] + 1.0)
    xw = jnp.dot(x_ref[...], w_ref[...], preferred_element_type=jnp.float32)
    y_ref[...] = xw * dis


def _fin_body(a0_ref, a1_ref, y_ref, d0_ref, d1_ref, b_ref, o_ref):
    dis = lax.rsqrt(d0_ref[...] + d1_ref[...] + 1.0)
    temp = dis * (a0_ref[...] + a1_ref[...] + y_ref[...]) + b_ref[...]
    o_ref[...] = temp + jnp.where(temp >= 0, temp, 0.01 * temp)


def kernel(x, edges, W, b):
    e32 = edges.astype(jnp.int32)
    src2d = e32[0].reshape(CT, K)
    dst2d = e32[1].reshape(CT, K)

    degp = _sc_degree(dst2d)
    d0 = degp[:NPAD].reshape(NPAD, 1)
    d1 = degp[NPAD:].reshape(NPAD, 1)

    y = pl.pallas_call(
        _mm_body,
        grid=(N_NODES // RBM,),
        in_specs=[
            pl.BlockSpec((RBM, CH), lambda i: (i, 0)),
            pl.BlockSpec((CH, CH), lambda i: (0, 0)),
            pl.BlockSpec((RBM, 1), lambda i: (i, 0)),
            pl.BlockSpec((RBM, 1), lambda i: (i, 0)),
        ],
        out_specs=pl.BlockSpec((RBM, CH), lambda i: (i, 0)),
        out_shape=jax.ShapeDtypeStruct((N_NODES, CH), jnp.float32),
    )(x, W, d0, d1)

    accp = _sc_edge_acc(y, src2d, dst2d)

    out = pl.pallas_call(
        _fin_body,
        grid=(N_NODES // RBM,),
        in_specs=[
            pl.BlockSpec((RBM, CH), lambda i: (i, 0)),
            pl.BlockSpec((RBM, CH), lambda i: (i, 0)),
            pl.BlockSpec((RBM, CH), lambda i: (i, 0)),
            pl.BlockSpec((RBM, 1), lambda i: (i, 0)),
            pl.BlockSpec((RBM, 1), lambda i: (i, 0)),
            pl.BlockSpec((1, CH), lambda i: (0, 0)),
        ],
        out_specs=pl.BlockSpec((RBM, CH), lambda i: (i, 0)),
        out_shape=jax.ShapeDtypeStruct((N_NODES, CH), jnp.float32),
    )(accp[:NPAD], accp[NPAD:], y, d0, d1, b.reshape(1, CH))

    return out
